# baseline (speedup 1.0000x reference)
import jax
import jax.numpy as jnp
from jax import lax
from jax.experimental import pallas as pl
from jax.experimental.pallas import tpu as pltpu
from jax.experimental.pallas import tpu_sc as plsc

N = 10000
E = 160000
D = 256
NPAD = 10240
EPAD = 163840
CW = 128
CHUNKS = EPAD // CW
ACPT = CHUNKS // 32
RANGE = NPAD // 32
RCAP = 256
ZROW = N + 16
F32 = jnp.float32
I32 = jnp.int32

_MESH = plsc.VectorSubcoreMesh(core_axis_name="c", subcore_axis_name="s")


def _deg_body(colp_hbm, deg0_hbm, deg1_hbm, colv, acc, tmp, outbuf,
              stage_sh, sem):
    c = lax.axis_index("c")
    s = lax.axis_index("s")

    def _z(i, _):
        acc[pl.ds(i * 16, 16)] = jnp.zeros((16,), F32)
        return 0
    lax.fori_loop(0, NPAD // 16, _z, 0)

    pltpu.sync_copy(colp_hbm.at[pl.ds((c * 16 + s) * ACPT, ACPT)], colv)

    lanes = lax.iota(I32, 16)

    def _h(i, _):
        j = lax.shift_right_logical(i, 3)
        k = lax.bitwise_and(i, 7)
        v = colv[j, pl.ds(k * 16, 16)]
        for m in range(16):
            e = v[m]
            l = lax.bitwise_and(e, 15)
            base = e - l
            oh = jnp.where(lanes == l, 1.0, 0.0).astype(F32)
            acc[pl.ds(base, 16)] = acc[pl.ds(base, 16)] + oh
        return 0
    lax.fori_loop(0, ACPT * 8, _h, 0)

    pltpu.sync_copy(acc, stage_sh.at[s])
    plsc.subcore_barrier()

    for t in range(16):
        pltpu.sync_copy(stage_sh.at[t, pl.ds(s * 640, 640)], tmp.at[t])

    def _sum(g, _):
        v = tmp[0, pl.ds(g * 16, 16)]
        for t in range(1, 16):
            v = v + tmp[t, pl.ds(g * 16, 16)]
        outbuf[g, :] = v
        return 0
    lax.fori_loop(0, 40, _sum, 0)

    @pl.when(c == 0)
    def _():
        pltpu.sync_copy(outbuf, deg0_hbm.at[pl.ds(s * 40, 40)])

    @pl.when(c == 1)
    def _():
        pltpu.sync_copy(outbuf, deg1_hbm.at[pl.ds(s * 40, 40)])


_deg_call = pl.kernel(
    _deg_body,
    out_type=(jax.ShapeDtypeStruct((NPAD // 16, 16), F32),
              jax.ShapeDtypeStruct((NPAD // 16, 16), F32)),
    mesh=_MESH,
    scratch_types=[
        pltpu.VMEM((ACPT, CW), I32),
        pltpu.VMEM((NPAD,), F32),
        pltpu.VMEM((16, 640), F32),
        pltpu.VMEM((40, 16), F32),
        pltpu.VMEM_SHARED((16, NPAD), F32),
        pltpu.SemaphoreType.DMA,
    ],
)


def _bin_body(pe_hbm, runs_hbm, counts_hbm, pev, runs, cnt, sem):
    c = lax.axis_index("c")
    s = lax.axis_index("s")
    wid = c * 16 + s
    lanes = lax.iota(I32, 16)

    pltpu.sync_copy(pe_hbm.at[pl.ds(wid * ACPT, ACPT)], pev)

    null_vec = RANGE * 16384 + ZROW + lanes

    def _fill(i, _):
        runs[pl.ds(i * 16, 16)] = null_vec
        return 0
    lax.fori_loop(0, 32 * RCAP // 16, _fill, 0)

    def _cz(i, _):
        cnt[pl.ds(i * 16, 16)] = jnp.zeros((16,), F32)
        return 0
    lax.fori_loop(0, 32, _cz, 0)

    one0 = jnp.where(lanes == 0, 1.0, 0.0).astype(F32)

    def _scan(i, _):
        j = lax.shift_right_logical(i, 3)
        k = lax.bitwise_and(i, 7)
        v = pev[j, pl.ds(k * 16, 16)]
        for m in range(16):
            p = v[m]
            col = lax.shift_right_logical(p, 14)
            row = lax.bitwise_and(p, 16383)
            o = lax.shift_right_logical(col * 6554, 21)
            lc = col - o * 320
            pnew = lc * 16384 + row
            wv = cnt[pl.ds(o * 16, 16)]
            c_val = jnp.minimum(wv[0].astype(I32), RCAP - 1)
            off = o * RCAP + c_val
            lane_t = lax.bitwise_and(off, 15)
            base = off - lane_t
            rv = runs[pl.ds(base, 16)]
            runs[pl.ds(base, 16)] = jnp.where(lanes == lane_t, pnew, rv)
            cnt[pl.ds(o * 16, 16)] = wv + one0
        return 0

    lax.fori_loop(0, ACPT * 8, _scan, 0)

    for o in range(32):
        pltpu.sync_copy(cnt.at[pl.ds(o * 16, 16)],
                        counts_hbm.at[pl.ds((o * 32 + wid) * 16, 16)])

    pltpu.sync_copy(runs, runs_hbm.at[pl.ds(wid * 32 * RCAP, 32 * RCAP)])


_bin_call = pl.kernel(
    _bin_body,
    out_type=(jax.ShapeDtypeStruct((32 * 32 * RCAP,), I32),
              jax.ShapeDtypeStruct((32 * 32 * 16,), F32)),
    mesh=_MESH,
    scratch_types=[
        pltpu.VMEM((ACPT, CW), I32),
        pltpu.VMEM((32 * RCAP,), I32),
        pltpu.VMEM((32 * 16,), F32),
        pltpu.SemaphoreType.DMA,
    ],
)


def _agg_body(hp_hbm, runs_hbm, counts_hbm, out_hbm,
              cv, pkb, rowv, msg, acc, sem):
    c = lax.axis_index("c")
    s = lax.axis_index("s")
    w = c * 16 + s
    lanes = lax.iota(I32, 16)

    def _z2(i, _):
        r = lax.shift_right_logical(i, 4)
        k = lax.bitwise_and(i, 15)
        acc[r, pl.ds(k * 16, 16)] = jnp.zeros((16,), F32)
        return 0
    lax.fori_loop(0, (RANGE + 1) * 16, _z2, 0)

    pltpu.sync_copy(counts_hbm.at[pl.ds(w * 512, 512)], cv)

    def _per_tile(t, _):
        c_t = cv[pl.ds(t * 16, 16)][0].astype(I32)
        for q in range(2):
            start = q * 128
            nv = jnp.maximum(
                0, jnp.minimum(8, lax.shift_right_logical(
                    jnp.maximum(c_t - start + 15, 0), 4)))

            @pl.when(nv > 0)
            def _():
                base = (t * 32 + w) * RCAP + start
                pltpu.sync_copy(runs_hbm.at[pl.ds(base, 128)], pkb)
                for g in range(8):
                    pv = pkb[pl.ds(g * 16, 16)]
                    rowv[0, pl.ds(g * 16, 16)] = lax.bitwise_and(pv, 16383)
                pltpu.async_copy(hp_hbm.at[rowv.at[0]], msg, sem).wait()

                def _grp(g, _):
                    pv = pkb[pl.ds(g * 16, 16)]
                    lcv = lax.shift_right_logical(pv, 14)
                    for m in range(16):
                        lc = lcv[m]
                        r = g * 16 + m
                        for k in range(16):
                            acc[lc, pl.ds(k * 16, 16)] = (
                                acc[lc, pl.ds(k * 16, 16)]
                                + msg[r, pl.ds(k * 16, 16)])
                    return 0
                lax.fori_loop(0, nv, _grp, 0)
        return 0
    lax.fori_loop(0, 32, _per_tile, 0)

    pltpu.sync_copy(acc.at[pl.ds(0, RANGE)],
                    out_hbm.at[pl.ds(w * RANGE, RANGE)])


_agg_call = pl.kernel(
    _agg_body,
    out_type=jax.ShapeDtypeStruct((NPAD, D), F32),
    mesh=_MESH,
    scratch_types=[
        pltpu.VMEM((512,), F32),
        pltpu.VMEM((128,), I32),
        pltpu.VMEM((1, 128), I32),
        pltpu.VMEM((CW, D), F32),
        pltpu.VMEM((RANGE + 1, D), F32),
        pltpu.SemaphoreType.DMA,
    ],
)


_BR = 1280
_GRID = NPAD // _BR


def _tc1_body(x_ref, w_ref, d0_ref, d1_ref, h_ref, dis_ref):
    deg = d0_ref[...] + d1_ref[...]
    di = lax.rsqrt(deg + 1.0)
    h = jnp.dot(x_ref[...], w_ref[...], preferred_element_type=F32)
    h_ref[...] = h * di
    dis_ref[...] = di


def _tc2_body(a_ref, hp_ref, dis_ref, b_ref, w_ref, out_ref):
    i = pl.program_id(0)
    rows = lax.broadcasted_iota(I32, (_BR, 1), 0) + i * _BR
    rmask = (rows < N).astype(F32)
    di = dis_ref[...]
    t = (a_ref[...] + hp_ref[...]) * di + b_ref[...]
    h = jnp.maximum(t, 0.0) * rmask
    out_ref[...] = jnp.dot(h, w_ref[...], preferred_element_type=F32) * di


def _tc3_body(a_ref, hp_ref, dis_ref, b_ref, w_ref, bfc_ref, out_ref):
    di = dis_ref[...]
    t = (a_ref[...] + hp_ref[...]) * di + b_ref[...]
    h = jnp.maximum(t, 0.0)
    out_ref[...] = (jnp.dot(h, w_ref[...], preferred_element_type=F32)
                    + bfc_ref[...])


def _rows_spec(width):
    return pl.BlockSpec((_BR, width), lambda i: (i, 0))


def _full_spec(shape):
    return pl.BlockSpec(shape, lambda i: (0,) * len(shape))


_tc1_call = pl.pallas_call(
    _tc1_body,
    grid=(_GRID,),
    in_specs=[_rows_spec(D), _full_spec((D, D)),
              _rows_spec(1), _rows_spec(1)],
    out_specs=(_rows_spec(D), _rows_spec(1)),
    out_shape=(jax.ShapeDtypeStruct((NPAD, D), F32),
               jax.ShapeDtypeStruct((NPAD, 1), F32)),
)

_tc2_call = pl.pallas_call(
    _tc2_body,
    grid=(_GRID,),
    in_specs=[_rows_spec(D), _rows_spec(D), _rows_spec(1),
              _full_spec((1, D)), _full_spec((D, D))],
    out_specs=_rows_spec(D),
    out_shape=jax.ShapeDtypeStruct((NPAD, D), F32),
)

_tc3_call = pl.pallas_call(
    _tc3_body,
    grid=(_GRID,),
    in_specs=[_rows_spec(D), _rows_spec(D), _rows_spec(1),
              _full_spec((1, D)), _full_spec((D, D)), _full_spec((1, D))],
    out_specs=_rows_spec(D),
    out_shape=jax.ShapeDtypeStruct((NPAD, D), F32),
)


def kernel(x, edge_index, W1, b1, W2, b2, Wfc, bfc):
    ei = edge_index.astype(I32)
    row = ei[0]
    col = ei[1]
    colp = jnp.concatenate(
        [col, jnp.full((EPAD - E,), NPAD - 1, I32)]).reshape(CHUNKS, CW)
    pe = jnp.concatenate(
        [col * 16384 + row,
         jnp.full((EPAD - E,), (NPAD - 1) * 16384, I32)]).reshape(CHUNKS, CW)
    xp = jnp.concatenate([x, jnp.zeros((NPAD - N, D), F32)])

    deg0, deg1 = _deg_call(colp)
    runs, counts = _bin_call(pe)
    h1p, dis = _tc1_call(xp, W1, deg0.reshape(NPAD, 1), deg1.reshape(NPAD, 1))
    agg1 = _agg_call(h1p, runs, counts)
    h2p = _tc2_call(agg1, h1p, dis, b1.reshape(1, D), W2)
    agg2 = _agg_call(h2p, runs, counts)
    out = _tc3_call(agg2, h2p, dis, b2.reshape(1, D), Wfc, bfc.reshape(1, D))
    return out[:N]


# speedup vs baseline: 1.0330x; 1.0330x over previous
import jax
import jax.numpy as jnp
from jax import lax
from jax.experimental import pallas as pl
from jax.experimental.pallas import tpu as pltpu
from jax.experimental.pallas import tpu_sc as plsc

N = 10000
E = 160000
D = 256
NPAD = 10240
EPAD = 163840
CW = 128
CHUNKS = EPAD // CW
ACPT = CHUNKS // 32
RANGE = NPAD // 32
RCAP = 256
ZROW = N + 16
F32 = jnp.float32
I32 = jnp.int32

_MESH = plsc.VectorSubcoreMesh(core_axis_name="c", subcore_axis_name="s")


def _deg_body(colp_hbm, deg0_hbm, deg1_hbm, colv, acc, tmp, outbuf,
              stage_sh, sem):
    c = lax.axis_index("c")
    s = lax.axis_index("s")

    def _z(i, _):
        acc[pl.ds(i * 16, 16)] = jnp.zeros((16,), F32)
        return 0
    lax.fori_loop(0, NPAD // 16, _z, 0)

    pltpu.sync_copy(colp_hbm.at[pl.ds((c * 16 + s) * ACPT, ACPT)], colv)

    lanes = lax.iota(I32, 16)

    def _h(i, _):
        j = lax.shift_right_logical(i, 3)
        k = lax.bitwise_and(i, 7)
        v = colv[j, pl.ds(k * 16, 16)]
        for m in range(16):
            e = v[m]
            l = lax.bitwise_and(e, 15)
            base = e - l
            oh = jnp.where(lanes == l, 1.0, 0.0).astype(F32)
            acc[pl.ds(base, 16)] = acc[pl.ds(base, 16)] + oh
        return 0
    lax.fori_loop(0, ACPT * 8, _h, 0)

    pltpu.sync_copy(acc, stage_sh.at[s])
    plsc.subcore_barrier()

    for t in range(16):
        pltpu.sync_copy(stage_sh.at[t, pl.ds(s * 640, 640)], tmp.at[t])

    def _sum(g, _):
        v = tmp[0, pl.ds(g * 16, 16)]
        for t in range(1, 16):
            v = v + tmp[t, pl.ds(g * 16, 16)]
        outbuf[g, :] = v
        return 0
    lax.fori_loop(0, 40, _sum, 0)

    @pl.when(c == 0)
    def _():
        pltpu.sync_copy(outbuf, deg0_hbm.at[pl.ds(s * 40, 40)])

    @pl.when(c == 1)
    def _():
        pltpu.sync_copy(outbuf, deg1_hbm.at[pl.ds(s * 40, 40)])


_deg_call = pl.kernel(
    _deg_body,
    out_type=(jax.ShapeDtypeStruct((NPAD // 16, 16), F32),
              jax.ShapeDtypeStruct((NPAD // 16, 16), F32)),
    mesh=_MESH,
    scratch_types=[
        pltpu.VMEM((ACPT, CW), I32),
        pltpu.VMEM((NPAD,), F32),
        pltpu.VMEM((16, 640), F32),
        pltpu.VMEM((40, 16), F32),
        pltpu.VMEM_SHARED((16, NPAD), F32),
        pltpu.SemaphoreType.DMA,
    ],
)


def _bin_body(pe_hbm, runs_hbm, counts_hbm, pev, runs, cnt, sem):
    c = lax.axis_index("c")
    s = lax.axis_index("s")
    wid = c * 16 + s
    lanes = lax.iota(I32, 16)

    pltpu.sync_copy(pe_hbm.at[pl.ds(wid * ACPT, ACPT)], pev)

    null_vec = RANGE * 16384 + ZROW + lanes

    def _fill(i, _):
        runs[pl.ds(i * 16, 16)] = null_vec
        return 0
    lax.fori_loop(0, 32 * RCAP // 16, _fill, 0)

    def _cz(i, _):
        cnt[pl.ds(i * 16, 16)] = jnp.zeros((16,), F32)
        return 0
    lax.fori_loop(0, 32, _cz, 0)

    one0 = jnp.where(lanes == 0, 1.0, 0.0).astype(F32)

    def _scan(i, _):
        j = lax.shift_right_logical(i, 3)
        k = lax.bitwise_and(i, 7)
        v = pev[j, pl.ds(k * 16, 16)]
        for m in range(16):
            p = v[m]
            col = lax.shift_right_logical(p, 14)
            row = lax.bitwise_and(p, 16383)
            o = lax.shift_right_logical(col * 6554, 21)
            lc = col - o * 320
            pnew = lc * 16384 + row
            wv = cnt[pl.ds(o * 16, 16)]
            c_val = jnp.minimum(wv[0].astype(I32), RCAP - 1)
            off = o * RCAP + c_val
            lane_t = lax.bitwise_and(off, 15)
            base = off - lane_t
            rv = runs[pl.ds(base, 16)]
            runs[pl.ds(base, 16)] = jnp.where(lanes == lane_t, pnew, rv)
            cnt[pl.ds(o * 16, 16)] = wv + one0
        return 0

    lax.fori_loop(0, ACPT * 8, _scan, 0)

    for o in range(32):
        pltpu.sync_copy(cnt.at[pl.ds(o * 16, 16)],
                        counts_hbm.at[pl.ds((o * 32 + wid) * 16, 16)])

    for o in range(32):
        pltpu.sync_copy(runs.at[pl.ds(o * RCAP, RCAP)],
                        runs_hbm.at[pl.ds((o * 32 + wid) * RCAP, RCAP)])


_bin_call = pl.kernel(
    _bin_body,
    out_type=(jax.ShapeDtypeStruct((32 * 32 * RCAP,), I32),
              jax.ShapeDtypeStruct((32 * 32 * 16,), F32)),
    mesh=_MESH,
    scratch_types=[
        pltpu.VMEM((ACPT, CW), I32),
        pltpu.VMEM((32 * RCAP,), I32),
        pltpu.VMEM((32 * 16,), F32),
        pltpu.SemaphoreType.DMA,
    ],
)


def _agg_body(hp_hbm, runs_hbm, counts_hbm, out_hbm,
              cv, pkv, rowv, msg, acc, sem):
    c = lax.axis_index("c")
    s = lax.axis_index("s")
    w = c * 16 + s
    lanes = lax.iota(I32, 16)

    def _z2(i, _):
        r = lax.shift_right_logical(i, 4)
        k = lax.bitwise_and(i, 15)
        acc[r, pl.ds(k * 16, 16)] = jnp.zeros((16,), F32)
        return 0
    lax.fori_loop(0, (RANGE + 1) * 16, _z2, 0)

    pltpu.sync_copy(counts_hbm.at[pl.ds(w * 512, 512)], cv)
    pltpu.sync_copy(runs_hbm.at[pl.ds(w * 32 * RCAP, 32 * RCAP)], pkv)

    def _per_tile(t, _):
        c_t = cv[pl.ds(t * 16, 16)][0].astype(I32)
        for q in range(2):
            start = q * 128
            nv = jnp.maximum(
                0, jnp.minimum(8, lax.shift_right_logical(
                    jnp.maximum(c_t - start + 15, 0), 4)))

            @pl.when(nv > 0)
            def _():
                base = t * RCAP + start
                for g in range(8):
                    pv = pkv[pl.ds(base + g * 16, 16)]
                    rowv[0, pl.ds(g * 16, 16)] = lax.bitwise_and(pv, 16383)
                pltpu.async_copy(hp_hbm.at[rowv.at[0]], msg, sem).wait()

                def _grp(g, _):
                    pv = pkv[pl.ds(base + g * 16, 16)]
                    lcv = lax.shift_right_logical(pv, 14)
                    for m in range(16):
                        lc = lcv[m]
                        r = g * 16 + m
                        for k in range(16):
                            acc[lc, pl.ds(k * 16, 16)] = (
                                acc[lc, pl.ds(k * 16, 16)]
                                + msg[r, pl.ds(k * 16, 16)])
                    return 0
                lax.fori_loop(0, nv, _grp, 0)
        return 0
    lax.fori_loop(0, 32, _per_tile, 0)

    pltpu.sync_copy(acc.at[pl.ds(0, RANGE)],
                    out_hbm.at[pl.ds(w * RANGE, RANGE)])


_agg_call = pl.kernel(
    _agg_body,
    out_type=jax.ShapeDtypeStruct((NPAD, D), F32),
    mesh=_MESH,
    scratch_types=[
        pltpu.VMEM((512,), F32),
        pltpu.VMEM((32 * RCAP,), I32),
        pltpu.VMEM((1, 128), I32),
        pltpu.VMEM((CW, D), F32),
        pltpu.VMEM((RANGE + 1, D), F32),
        pltpu.SemaphoreType.DMA,
    ],
)


_BR = 1280
_GRID = NPAD // _BR


def _tc1_body(x_ref, w_ref, d0_ref, d1_ref, h_ref, dis_ref):
    deg = d0_ref[...] + d1_ref[...]
    di = lax.rsqrt(deg + 1.0)
    h = jnp.dot(x_ref[...], w_ref[...], preferred_element_type=F32)
    h_ref[...] = h * di
    dis_ref[...] = di


def _tc2_body(a_ref, hp_ref, dis_ref, b_ref, w_ref, out_ref):
    i = pl.program_id(0)
    rows = lax.broadcasted_iota(I32, (_BR, 1), 0) + i * _BR
    rmask = (rows < N).astype(F32)
    di = dis_ref[...]
    t = (a_ref[...] + hp_ref[...]) * di + b_ref[...]
    h = jnp.maximum(t, 0.0) * rmask
    out_ref[...] = jnp.dot(h, w_ref[...], preferred_element_type=F32) * di


def _tc3_body(a_ref, hp_ref, dis_ref, b_ref, w_ref, bfc_ref, out_ref):
    di = dis_ref[...]
    t = (a_ref[...] + hp_ref[...]) * di + b_ref[...]
    h = jnp.maximum(t, 0.0)
    out_ref[...] = (jnp.dot(h, w_ref[...], preferred_element_type=F32)
                    + bfc_ref[...])


def _rows_spec(width):
    return pl.BlockSpec((_BR, width), lambda i: (i, 0))


def _full_spec(shape):
    return pl.BlockSpec(shape, lambda i: (0,) * len(shape))


_tc1_call = pl.pallas_call(
    _tc1_body,
    grid=(_GRID,),
    in_specs=[_rows_spec(D), _full_spec((D, D)),
              _rows_spec(1), _rows_spec(1)],
    out_specs=(_rows_spec(D), _rows_spec(1)),
    out_shape=(jax.ShapeDtypeStruct((NPAD, D), F32),
               jax.ShapeDtypeStruct((NPAD, 1), F32)),
)

_tc2_call = pl.pallas_call(
    _tc2_body,
    grid=(_GRID,),
    in_specs=[_rows_spec(D), _rows_spec(D), _rows_spec(1),
              _full_spec((1, D)), _full_spec((D, D))],
    out_specs=_rows_spec(D),
    out_shape=jax.ShapeDtypeStruct((NPAD, D), F32),
)

_tc3_call = pl.pallas_call(
    _tc3_body,
    grid=(_GRID,),
    in_specs=[_rows_spec(D), _rows_spec(D), _rows_spec(1),
              _full_spec((1, D)), _full_spec((D, D)), _full_spec((1, D))],
    out_specs=_rows_spec(D),
    out_shape=jax.ShapeDtypeStruct((NPAD, D), F32),
)


def kernel(x, edge_index, W1, b1, W2, b2, Wfc, bfc):
    ei = edge_index.astype(I32)
    row = ei[0]
    col = ei[1]
    colp = jnp.concatenate(
        [col, jnp.full((EPAD - E,), NPAD - 1, I32)]).reshape(CHUNKS, CW)
    pe = jnp.concatenate(
        [col * 16384 + row,
         jnp.full((EPAD - E,), (NPAD - 1) * 16384, I32)]).reshape(CHUNKS, CW)
    xp = jnp.concatenate([x, jnp.zeros((NPAD - N, D), F32)])

    deg0, deg1 = _deg_call(colp)
    runs, counts = _bin_call(pe)
    h1p, dis = _tc1_call(xp, W1, deg0.reshape(NPAD, 1), deg1.reshape(NPAD, 1))
    agg1 = _agg_call(h1p, runs, counts)
    h2p = _tc2_call(agg1, h1p, dis, b1.reshape(1, D), W2)
    agg2 = _agg_call(h2p, runs, counts)
    out = _tc3_call(agg2, h2p, dis, b2.reshape(1, D), Wfc, bfc.reshape(1, D))
    return out[:N]


# speedup vs baseline: 1.2015x; 1.1631x over previous
import jax
import jax.numpy as jnp
from jax import lax
from jax.experimental import pallas as pl
from jax.experimental.pallas import tpu as pltpu
from jax.experimental.pallas import tpu_sc as plsc

N = 10000
E = 160000
D = 256
NPAD = 10240
EPAD = 163840
CW = 128
CHUNKS = EPAD // CW
ACPT = CHUNKS // 32
RANGE = NPAD // 32
RCAP = 256
ZROW = N + 16
F32 = jnp.float32
I32 = jnp.int32

_MESH = plsc.VectorSubcoreMesh(core_axis_name="c", subcore_axis_name="s")


def _deg_body(colp_hbm, deg0_hbm, deg1_hbm, colv, acc, tmp, outbuf,
              stage_sh, sem):
    c = lax.axis_index("c")
    s = lax.axis_index("s")

    def _z(i, _):
        acc[pl.ds(i * 16, 16)] = jnp.zeros((16,), F32)
        return 0
    lax.fori_loop(0, NPAD // 16, _z, 0)

    pltpu.sync_copy(colp_hbm.at[pl.ds((c * 16 + s) * ACPT, ACPT)], colv)

    lanes = lax.iota(I32, 16)

    def _h(i, _):
        j = lax.shift_right_logical(i, 3)
        k = lax.bitwise_and(i, 7)
        v = colv[j, pl.ds(k * 16, 16)]
        for m in range(16):
            e = v[m]
            l = lax.bitwise_and(e, 15)
            base = e - l
            oh = jnp.where(lanes == l, 1.0, 0.0).astype(F32)
            acc[pl.ds(base, 16)] = acc[pl.ds(base, 16)] + oh
        return 0
    lax.fori_loop(0, ACPT * 8, _h, 0)

    pltpu.sync_copy(acc, stage_sh.at[s])
    plsc.subcore_barrier()

    for t in range(16):
        pltpu.sync_copy(stage_sh.at[t, pl.ds(s * 640, 640)], tmp.at[t])

    def _sum(g, _):
        v = tmp[0, pl.ds(g * 16, 16)]
        for t in range(1, 16):
            v = v + tmp[t, pl.ds(g * 16, 16)]
        outbuf[g, :] = v
        return 0
    lax.fori_loop(0, 40, _sum, 0)

    @pl.when(c == 0)
    def _():
        pltpu.sync_copy(outbuf, deg0_hbm.at[pl.ds(s * 40, 40)])

    @pl.when(c == 1)
    def _():
        pltpu.sync_copy(outbuf, deg1_hbm.at[pl.ds(s * 40, 40)])


_deg_call = pl.kernel(
    _deg_body,
    out_type=(jax.ShapeDtypeStruct((NPAD // 16, 16), F32),
              jax.ShapeDtypeStruct((NPAD // 16, 16), F32)),
    mesh=_MESH,
    scratch_types=[
        pltpu.VMEM((ACPT, CW), I32),
        pltpu.VMEM((NPAD,), F32),
        pltpu.VMEM((16, 640), F32),
        pltpu.VMEM((40, 16), F32),
        pltpu.VMEM_SHARED((16, NPAD), F32),
        pltpu.SemaphoreType.DMA,
    ],
)


def _bin_body(pe_hbm, runs_hbm, counts_hbm, pev, runs, cnt, sem):
    c = lax.axis_index("c")
    s = lax.axis_index("s")
    wid = c * 16 + s
    lanes = lax.iota(I32, 16)

    pltpu.sync_copy(pe_hbm.at[pl.ds(wid * ACPT, ACPT)], pev)

    null_vec = RANGE * 16384 + ZROW + lanes

    def _fill(i, _):
        runs[pl.ds(i * 16, 16)] = null_vec
        return 0
    lax.fori_loop(0, 32 * RCAP // 16, _fill, 0)

    def _cz(i, _):
        cnt[pl.ds(i * 16, 16)] = jnp.zeros((16,), F32)
        return 0
    lax.fori_loop(0, 32, _cz, 0)

    one0 = jnp.where(lanes == 0, 1.0, 0.0).astype(F32)

    def _scan(i, _):
        j = lax.shift_right_logical(i, 3)
        k = lax.bitwise_and(i, 7)
        v = pev[j, pl.ds(k * 16, 16)]
        for m in range(16):
            p = v[m]
            col = lax.shift_right_logical(p, 14)
            row = lax.bitwise_and(p, 16383)
            o = lax.shift_right_logical(col * 6554, 21)
            lc = col - o * 320
            pnew = lc * 16384 + row
            wv = cnt[pl.ds(o * 16, 16)]
            c_val = jnp.minimum(wv[0].astype(I32), RCAP - 1)
            off = o * RCAP + c_val
            lane_t = lax.bitwise_and(off, 15)
            base = off - lane_t
            rv = runs[pl.ds(base, 16)]
            runs[pl.ds(base, 16)] = jnp.where(lanes == lane_t, pnew, rv)
            cnt[pl.ds(o * 16, 16)] = wv + one0
        return 0

    lax.fori_loop(0, ACPT * 8, _scan, 0)

    for o in range(32):
        pltpu.sync_copy(cnt.at[pl.ds(o * 16, 16)],
                        counts_hbm.at[pl.ds((o * 32 + wid) * 16, 16)])

    for o in range(32):
        pltpu.sync_copy(runs.at[pl.ds(o * RCAP, RCAP)],
                        runs_hbm.at[pl.ds((o * 32 + wid) * RCAP, RCAP)])


_bin_call = pl.kernel(
    _bin_body,
    out_type=(jax.ShapeDtypeStruct((32 * 32 * RCAP,), I32),
              jax.ShapeDtypeStruct((32 * 32 * 16,), F32)),
    mesh=_MESH,
    scratch_types=[
        pltpu.VMEM((ACPT, CW), I32),
        pltpu.VMEM((32 * RCAP,), I32),
        pltpu.VMEM((32 * 16,), F32),
        pltpu.SemaphoreType.DMA,
    ],
)


def _agg_body(hp_hbm, runs_hbm, counts_hbm, out_hbm,
              cv, pkv, rowv, msg, acc, sem):
    c = lax.axis_index("c")
    s = lax.axis_index("s")
    w = c * 16 + s
    lanes = lax.iota(I32, 16)

    def _z2(i, _):
        r = lax.shift_right_logical(i, 4)
        k = lax.bitwise_and(i, 15)
        acc[r, pl.ds(k * 16, 16)] = jnp.zeros((16,), F32)
        return 0
    lax.fori_loop(0, (RANGE + 1) * 16, _z2, 0)

    pltpu.sync_copy(counts_hbm.at[pl.ds(w * 512, 512)], cv)
    pltpu.sync_copy(runs_hbm.at[pl.ds(w * 32 * RCAP, 32 * RCAP)], pkv)

    def _per_tile(t, _):
        c_t = cv[pl.ds(t * 16, 16)][0].astype(I32)
        for q in range(2):
            start = q * 128
            nv = jnp.maximum(
                0, jnp.minimum(8, lax.shift_right_logical(
                    jnp.maximum(c_t - start + 15, 0), 4)))

            @pl.when(nv > 0)
            def _():
                base = t * RCAP + start
                for g in range(8):
                    pv = pkv[pl.ds(base + g * 16, 16)]
                    rowv[0, pl.ds(g * 16, 16)] = lax.bitwise_and(pv, 16383)
                pltpu.async_copy(hp_hbm.at[rowv.at[0]], msg, sem).wait()

                def _grp(g, _):
                    pv = pkv[pl.ds(base + g * 16, 16)]
                    lcv = lax.shift_right_logical(pv, 14)
                    for m in range(16):
                        lc = lcv[m]
                        r = g * 16 + m
                        for k in range(16):
                            plsc.addupdate(acc.at[lc, pl.ds(k * 16, 16)],
                                           msg[r, pl.ds(k * 16, 16)])
                    return 0
                lax.fori_loop(0, nv, _grp, 0)
        return 0
    lax.fori_loop(0, 32, _per_tile, 0)

    pltpu.sync_copy(acc.at[pl.ds(0, RANGE)],
                    out_hbm.at[pl.ds(w * RANGE, RANGE)])


_agg_call = pl.kernel(
    _agg_body,
    out_type=jax.ShapeDtypeStruct((NPAD, D), F32),
    mesh=_MESH,
    scratch_types=[
        pltpu.VMEM((512,), F32),
        pltpu.VMEM((32 * RCAP,), I32),
        pltpu.VMEM((1, 128), I32),
        pltpu.VMEM((CW, D), F32),
        pltpu.VMEM((RANGE + 1, D), F32),
        pltpu.SemaphoreType.DMA,
    ],
)


_BR = 1280
_GRID = NPAD // _BR


def _tc1_body(x_ref, w_ref, d0_ref, d1_ref, h_ref, dis_ref):
    deg = d0_ref[...] + d1_ref[...]
    di = lax.rsqrt(deg + 1.0)
    h = jnp.dot(x_ref[...], w_ref[...], preferred_element_type=F32)
    h_ref[...] = h * di
    dis_ref[...] = di


def _tc2_body(a_ref, hp_ref, dis_ref, b_ref, w_ref, out_ref):
    i = pl.program_id(0)
    rows = lax.broadcasted_iota(I32, (_BR, 1), 0) + i * _BR
    rmask = (rows < N).astype(F32)
    di = dis_ref[...]
    t = (a_ref[...] + hp_ref[...]) * di + b_ref[...]
    h = jnp.maximum(t, 0.0) * rmask
    out_ref[...] = jnp.dot(h, w_ref[...], preferred_element_type=F32) * di


def _tc3_body(a_ref, hp_ref, dis_ref, b_ref, w_ref, bfc_ref, out_ref):
    di = dis_ref[...]
    t = (a_ref[...] + hp_ref[...]) * di + b_ref[...]
    h = jnp.maximum(t, 0.0)
    out_ref[...] = (jnp.dot(h, w_ref[...], preferred_element_type=F32)
                    + bfc_ref[...])


def _rows_spec(width):
    return pl.BlockSpec((_BR, width), lambda i: (i, 0))


def _full_spec(shape):
    return pl.BlockSpec(shape, lambda i: (0,) * len(shape))


_tc1_call = pl.pallas_call(
    _tc1_body,
    grid=(_GRID,),
    in_specs=[_rows_spec(D), _full_spec((D, D)),
              _rows_spec(1), _rows_spec(1)],
    out_specs=(_rows_spec(D), _rows_spec(1)),
    out_shape=(jax.ShapeDtypeStruct((NPAD, D), F32),
               jax.ShapeDtypeStruct((NPAD, 1), F32)),
)

_tc2_call = pl.pallas_call(
    _tc2_body,
    grid=(_GRID,),
    in_specs=[_rows_spec(D), _rows_spec(D), _rows_spec(1),
              _full_spec((1, D)), _full_spec((D, D))],
    out_specs=_rows_spec(D),
    out_shape=jax.ShapeDtypeStruct((NPAD, D), F32),
)

_tc3_call = pl.pallas_call(
    _tc3_body,
    grid=(_GRID,),
    in_specs=[_rows_spec(D), _rows_spec(D), _rows_spec(1),
              _full_spec((1, D)), _full_spec((D, D)), _full_spec((1, D))],
    out_specs=_rows_spec(D),
    out_shape=jax.ShapeDtypeStruct((NPAD, D), F32),
)


def kernel(x, edge_index, W1, b1, W2, b2, Wfc, bfc):
    ei = edge_index.astype(I32)
    row = ei[0]
    col = ei[1]
    colp = jnp.concatenate(
        [col, jnp.full((EPAD - E,), NPAD - 1, I32)]).reshape(CHUNKS, CW)
    pe = jnp.concatenate(
        [col * 16384 + row,
         jnp.full((EPAD - E,), (NPAD - 1) * 16384, I32)]).reshape(CHUNKS, CW)
    xp = jnp.concatenate([x, jnp.zeros((NPAD - N, D), F32)])

    deg0, deg1 = _deg_call(colp)
    runs, counts = _bin_call(pe)
    h1p, dis = _tc1_call(xp, W1, deg0.reshape(NPAD, 1), deg1.reshape(NPAD, 1))
    agg1 = _agg_call(h1p, runs, counts)
    h2p = _tc2_call(agg1, h1p, dis, b1.reshape(1, D), W2)
    agg2 = _agg_call(h2p, runs, counts)
    out = _tc3_call(agg2, h2p, dis, b2.reshape(1, D), Wfc, bfc.reshape(1, D))
    return out[:N]
